# baseline (speedup 1.0000x reference)
import dataclasses
import functools
import math

import jax
import jax.numpy as jnp
from jax import lax
from jax.experimental import pallas as pl
from jax.experimental.pallas import tpu as pltpu
from jax.experimental.pallas import tpu_sc as plsc


def _dot(a, b):
    return jax.lax.dot_general(
        a, b, (((1,), (0,)), ((), ())), preferred_element_type=jnp.float32
    )


def _dot_t(a, b):
    return jax.lax.dot_general(
        a, b, (((1,), (1,)), ((), ())), preferred_element_type=jnp.float32
    )


def _ln_f32(x, g, b):
    m = jnp.mean(x, axis=1, keepdims=True)
    xc = x - m
    var = jnp.mean(xc * xc, axis=1, keepdims=True)
    return xc * jax.lax.rsqrt(var + 1e-5) * g + b


def _stage_a_kernel(vis_ref, text_ref, g1_ref, b1_ref, wq_ref, scale_ref,
                    x_ref, sim_ref, *, k_real):
    f32 = jnp.float32
    bf16 = jnp.bfloat16
    xv = vis_ref[0]
    x = _ln_f32(xv, g1_ref[0], b1_ref[0])
    x_ref[0] = x

    qpre = _dot(x.astype(bf16), wq_ref[...])
    qn = jnp.sqrt(jnp.sum(qpre * qpre, axis=1, keepdims=True))
    q = qpre / jnp.maximum(qn, 1e-12)

    t = text_ref[0]
    kn = jnp.sqrt(jnp.sum(t * t, axis=1, keepdims=True))
    k = t / jnp.maximum(kn, 1e-12)
    sim = _dot_t(q.astype(bf16), k.astype(bf16)) * scale_ref[0, 0]
    col = jax.lax.broadcasted_iota(jnp.int32, sim.shape, 1)
    sim_ref[0] = jnp.where(col < k_real, sim, jnp.asarray(-1e30, f32))


def _sc_merge(av, ai, bv, bi):
    rv = jnp.flip(bv, 0)
    ri = jnp.flip(bi, 0)
    ge = av >= rv
    mv = jnp.where(ge, av, rv)
    mi = jnp.where(ge, ai, ri)
    return plsc.sort_key_val(mv, mi, descending=True)


def _sc_topk_kernel(sim_hbm, w_hbm, i_hbm, sim_v, w_v, i_v, sem0, sem1, *,
                    rows_per_tile, chunk_rows, k_pad):
    f32 = jnp.float32
    i32 = jnp.int32
    n_chunks = k_pad // 16
    n_slabs = rows_per_tile // chunk_rows
    wid = lax.axis_index("s") * 2 + lax.axis_index("c")
    base = wid * rows_per_tile
    lane = lax.iota(i32, 16)
    sems = (sem0, sem1)

    def slab_copy(s, buf):
        return pltpu.make_async_copy(
            sim_hbm.at[pl.ds(base + s * chunk_rows, chunk_rows)],
            sim_v.at[buf], sems[buf])

    slab_copy(0, 0).start()
    for s in range(n_slabs):
        buf = s % 2
        slab_copy(s, buf).wait()
        if s + 1 < n_slabs:
            slab_copy(s + 1, 1 - buf).start()

        @plsc.parallel_loop(0, chunk_rows, unroll=4)
        def _(r):
            pairs = []
            for c in range(n_chunks):
                vals = sim_v[buf, r, pl.ds(c * 16, 16)]
                sv, si = plsc.sort_key_val(vals, lane + (c * 16),
                                           descending=True)
                pairs.append((sv, si))
            while len(pairs) > 1:
                nxt = []
                for j in range(0, len(pairs) - 1, 2):
                    nxt.append(_sc_merge(*pairs[j], *pairs[j + 1]))
                if len(pairs) % 2:
                    nxt.append(pairs[-1])
                pairs = nxt
            tv, ti = pairs[0]

            m0 = jnp.max(tv)
            e = jnp.exp(tv - jnp.full((16,), m0, f32))
            e = jnp.where(lane < 5, e, jnp.zeros((16,), f32))
            sm = jnp.sum(e)
            w_v[r] = e / jnp.full((16,), sm, f32)
            i_v[r] = ti

        pltpu.sync_copy(w_v, w_hbm.at[pl.ds(base + s * chunk_rows,
                                            chunk_rows)])
        pltpu.sync_copy(i_v, i_hbm.at[pl.ds(base + s * chunk_rows,
                                            chunk_rows)])


def _stage_b_kernel(x_ref, text_ref, w_ref, i_ref, wvt_ref, wg_ref, bg_ref,
                    g2_ref, b2_ref, w1t_ref, bf1_ref, w2t_ref, bf2_ref,
                    out_ref, *, k_pad):
    f32 = jnp.float32
    bf16 = jnp.bfloat16
    x = x_ref[0]
    n = x.shape[0]
    w = w_ref[0]
    ti = i_ref[0]

    col = jax.lax.broadcasted_iota(jnp.int32, (n, k_pad), 1)
    wmat = jnp.zeros((n, k_pad), f32)
    for m in range(5):
        sel = (col == ti[:, m:m + 1]).astype(f32)
        wmat = wmat + w[:, m:m + 1] * sel

    t = text_ref[0]
    v = _dot(t.astype(bf16), wvt_ref[...]).astype(bf16)
    aligned = _dot(wmat.astype(bf16), v)

    gl = jnp.sum(x * wg_ref[0], axis=1, keepdims=True) + bg_ref[0, 0]
    gate = jax.nn.sigmoid(gl)
    y = x + aligned * gate
    y2 = _ln_f32(y, g2_ref[0], b2_ref[0])

    h = _dot(y2.astype(bf16), w1t_ref[...]) + bf1_ref[0]
    h = 0.5 * h * (1.0 + jax.lax.erf(h * jnp.asarray(0.7071067811865476, f32)))
    out_ref[0] = y2 + _dot(h.astype(bf16), w2t_ref[...]) + bf2_ref[0]


def kernel(visual_features, text_features, g1, b1, Wq, Wv, logit_scale,
           Wg, bg, g2, b2, W1, bf1, W2, bf2):
    B, H, W, Cv = visual_features.shape
    K, Ct = text_features.shape[1], text_features.shape[2]
    N = H * W
    dff = W1.shape[0]
    K_pad = ((K + 127) // 128) * 128
    ROWS = B * N

    text_p = jnp.concatenate(
        [text_features,
         jnp.zeros((B, K_pad - K, Ct), text_features.dtype)], axis=1)
    vis = visual_features.reshape(B, N, Cv)
    wqt = Wq.T.astype(jnp.bfloat16)
    wvt = Wv.T.astype(jnp.bfloat16)
    w1t = W1.T.astype(jnp.bfloat16)
    w2t = W2.T.astype(jnp.bfloat16)
    scale = (jnp.exp(logit_scale) / math.sqrt(Ct)).reshape(1, 1)

    row = lambda a: a.reshape(1, -1)
    const = lambda shape: pl.BlockSpec(shape, lambda b: (0,) * len(shape))

    GROUPS = 1
    Bg = B // GROUPS
    ROWS_G = Bg * N

    info = plsc.get_sparse_core_info()
    n_tiles = info.num_cores * info.num_subcores
    rows_per_tile = ROWS_G // n_tiles
    chunk_rows = min(64, rows_per_tile)

    cp = pltpu.CompilerParams()
    if "needs_layout_passes" in pltpu.CompilerParams.__dataclass_fields__:
        cp = dataclasses.replace(cp, needs_layout_passes=False)
    mesh = plsc.VectorSubcoreMesh(core_axis_name="c", subcore_axis_name="s")
    sc_topk = pl.kernel(
        functools.partial(
            _sc_topk_kernel, rows_per_tile=rows_per_tile,
            chunk_rows=chunk_rows, k_pad=K_pad),
        mesh=mesh,
        out_type=[
            jax.ShapeDtypeStruct((ROWS_G, 16), jnp.float32),
            jax.ShapeDtypeStruct((ROWS_G, 16), jnp.int32),
        ],
        scratch_types=[
            pltpu.VMEM((2, chunk_rows, K_pad), jnp.float32),
            pltpu.VMEM((chunk_rows, 16), jnp.float32),
            pltpu.VMEM((chunk_rows, 16), jnp.int32),
            pltpu.SemaphoreType.DMA,
            pltpu.SemaphoreType.DMA,
        ],
        compiler_params=cp,
    )

    stage_a = pl.pallas_call(
        functools.partial(_stage_a_kernel, k_real=K),
        grid=(Bg,),
        in_specs=[
            pl.BlockSpec((1, N, Cv), lambda b: (b, 0, 0)),
            pl.BlockSpec((1, K_pad, Ct), lambda b: (b, 0, 0)),
            const((1, Cv)), const((1, Cv)),
            const((Cv, Ct)), const((1, 1)),
        ],
        out_specs=[
            pl.BlockSpec((1, N, Cv), lambda b: (b, 0, 0)),
            pl.BlockSpec((1, N, K_pad), lambda b: (b, 0, 0)),
        ],
        out_shape=[
            jax.ShapeDtypeStruct((Bg, N, Cv), jnp.float32),
            jax.ShapeDtypeStruct((Bg, N, K_pad), jnp.float32),
        ],
    )

    stage_b = pl.pallas_call(
        functools.partial(_stage_b_kernel, k_pad=K_pad),
        grid=(Bg,),
        in_specs=[
            pl.BlockSpec((1, N, Cv), lambda b: (b, 0, 0)),
            pl.BlockSpec((1, K_pad, Ct), lambda b: (b, 0, 0)),
            pl.BlockSpec((1, N, 16), lambda b: (b, 0, 0)),
            pl.BlockSpec((1, N, 16), lambda b: (b, 0, 0)),
            const((Ct, Cv)), const((1, Cv)), const((1, 1)),
            const((1, Cv)), const((1, Cv)),
            const((Cv, dff)), const((1, dff)),
            const((dff, Cv)), const((1, Cv)),
        ],
        out_specs=pl.BlockSpec((1, N, Cv), lambda b: (b, 0, 0)),
        out_shape=jax.ShapeDtypeStruct((Bg, N, Cv), jnp.float32),
    )

    sel = []
    for g in range(GROUPS):
        sl = slice(g * Bg, (g + 1) * Bg)
        x_g, sim_g = stage_a(vis[sl], text_p[sl], row(g1), row(b1), wqt,
                             scale)
        w_g, i_g = sc_topk(sim_g.reshape(ROWS_G, K_pad))
        sel.append((x_g, w_g, i_g))

    outs = []
    for g in range(GROUPS):
        sl = slice(g * Bg, (g + 1) * Bg)
        x_g, w_g, i_g = sel[g]
        outs.append(stage_b(
            x_g, text_p[sl], w_g.reshape(Bg, N, 16), i_g.reshape(Bg, N, 16),
            wvt, row(Wg), bg.reshape(1, 1), row(g2), row(b2),
            w1t, row(bf1), w2t, row(bf2),
        ))
    out = jnp.concatenate(outs, axis=0)
    return out.reshape(B, H, W, Cv)


# speedup vs baseline: 1.0333x; 1.0333x over previous
import dataclasses
import functools
import math

import jax
import jax.numpy as jnp
from jax import lax
from jax.experimental import pallas as pl
from jax.experimental.pallas import tpu as pltpu
from jax.experimental.pallas import tpu_sc as plsc


def _dot(a, b):
    return jax.lax.dot_general(
        a, b, (((1,), (0,)), ((), ())), preferred_element_type=jnp.float32
    )


def _dot_t(a, b):
    return jax.lax.dot_general(
        a, b, (((1,), (1,)), ((), ())), preferred_element_type=jnp.float32
    )


def _ln_f32(x, g, b):
    m = jnp.mean(x, axis=1, keepdims=True)
    xc = x - m
    var = jnp.mean(xc * xc, axis=1, keepdims=True)
    return xc * jax.lax.rsqrt(var + 1e-5) * g + b


def _stage_a_kernel(vis_ref, text_ref, g1_ref, b1_ref, wq_ref, scale_ref,
                    x_ref, sim_ref, *, k_real):
    f32 = jnp.float32
    bf16 = jnp.bfloat16
    xv = vis_ref[0]
    x = _ln_f32(xv, g1_ref[0], b1_ref[0])
    x_ref[0] = x.astype(jnp.bfloat16)

    qpre = _dot(x.astype(bf16), wq_ref[...])
    qn = jnp.sqrt(jnp.sum(qpre * qpre, axis=1, keepdims=True))
    q = qpre / jnp.maximum(qn, 1e-12)

    t = text_ref[0]
    kn = jnp.sqrt(jnp.sum(t * t, axis=1, keepdims=True))
    k = t / jnp.maximum(kn, 1e-12)
    sim = _dot_t(q.astype(bf16), k.astype(bf16)) * scale_ref[0, 0]
    col = jax.lax.broadcasted_iota(jnp.int32, sim.shape, 1)
    sim_ref[0] = jnp.where(col < k_real, sim, jnp.asarray(-1e30, f32))


def _sc_merge(av, ai, bv, bi):
    rv = jnp.flip(bv, 0)
    ri = jnp.flip(bi, 0)
    ge = av >= rv
    mv = jnp.where(ge, av, rv)
    mi = jnp.where(ge, ai, ri)
    return plsc.sort_key_val(mv, mi, descending=True)


def _sc_topk_kernel(sim_hbm, w_hbm, i_hbm, sim_v, w_v, i_v, sem0, sem1, *,
                    rows_per_tile, chunk_rows, k_pad):
    f32 = jnp.float32
    i32 = jnp.int32
    n_chunks = k_pad // 16
    n_slabs = rows_per_tile // chunk_rows
    wid = lax.axis_index("s") * 2 + lax.axis_index("c")
    base = wid * rows_per_tile
    lane = lax.iota(i32, 16)
    sems = (sem0, sem1)

    def slab_copy(s, buf):
        return pltpu.make_async_copy(
            sim_hbm.at[pl.ds(base + s * chunk_rows, chunk_rows)],
            sim_v.at[buf], sems[buf])

    slab_copy(0, 0).start()
    for s in range(n_slabs):
        buf = s % 2
        slab_copy(s, buf).wait()
        if s + 1 < n_slabs:
            slab_copy(s + 1, 1 - buf).start()

        @plsc.parallel_loop(0, chunk_rows, unroll=2)
        def _(r):
            pairs = []
            for c in range(n_chunks):
                vals = sim_v[buf, r, pl.ds(c * 16, 16)]
                sv, si = plsc.sort_key_val(vals, lane + (c * 16),
                                           descending=True)
                pairs.append((sv, si))
            while len(pairs) > 1:
                nxt = []
                for j in range(0, len(pairs) - 1, 2):
                    nxt.append(_sc_merge(*pairs[j], *pairs[j + 1]))
                if len(pairs) % 2:
                    nxt.append(pairs[-1])
                pairs = nxt
            tv, ti = pairs[0]

            m0 = jnp.max(tv)
            e = jnp.exp(tv - jnp.full((16,), m0, f32))
            e = jnp.where(lane < 5, e, jnp.zeros((16,), f32))
            sm = jnp.sum(e)
            w_v[r] = e / jnp.full((16,), sm, f32)
            i_v[r] = ti

        pltpu.sync_copy(w_v, w_hbm.at[pl.ds(base + s * chunk_rows,
                                            chunk_rows)])
        pltpu.sync_copy(i_v, i_hbm.at[pl.ds(base + s * chunk_rows,
                                            chunk_rows)])


def _stage_b_kernel(x_ref, text_ref, w_ref, i_ref, wvt_ref, wg_ref, bg_ref,
                    g2_ref, b2_ref, w1t_ref, bf1_ref, w2t_ref, bf2_ref,
                    out_ref, *, k_pad):
    f32 = jnp.float32
    bf16 = jnp.bfloat16
    x = x_ref[0].astype(f32)
    n = x.shape[0]
    w = w_ref[0]
    ti = i_ref[0]

    col = jax.lax.broadcasted_iota(jnp.int32, (n, k_pad), 1)
    wmat = jnp.zeros((n, k_pad), f32)
    for m in range(5):
        sel = (col == ti[:, m:m + 1]).astype(f32)
        wmat = wmat + w[:, m:m + 1] * sel

    t = text_ref[0]
    v = _dot(t.astype(bf16), wvt_ref[...]).astype(bf16)
    aligned = _dot(wmat.astype(bf16), v)

    gl = jnp.sum(x * wg_ref[0], axis=1, keepdims=True) + bg_ref[0, 0]
    gate = jax.nn.sigmoid(gl)
    y = x + aligned * gate
    y2 = _ln_f32(y, g2_ref[0], b2_ref[0])

    h = _dot(y2.astype(bf16), w1t_ref[...]) + bf1_ref[0]
    h = 0.5 * h * (1.0 + jax.lax.erf(h * jnp.asarray(0.7071067811865476, f32)))
    out_ref[0] = y2 + _dot(h.astype(bf16), w2t_ref[...]) + bf2_ref[0]


def kernel(visual_features, text_features, g1, b1, Wq, Wv, logit_scale,
           Wg, bg, g2, b2, W1, bf1, W2, bf2):
    B, H, W, Cv = visual_features.shape
    K, Ct = text_features.shape[1], text_features.shape[2]
    N = H * W
    dff = W1.shape[0]
    K_pad = ((K + 127) // 128) * 128
    ROWS = B * N

    text_p = jnp.concatenate(
        [text_features,
         jnp.zeros((B, K_pad - K, Ct), text_features.dtype)], axis=1)
    vis = visual_features.reshape(B, N, Cv)
    wqt = Wq.T.astype(jnp.bfloat16)
    wvt = Wv.T.astype(jnp.bfloat16)
    w1t = W1.T.astype(jnp.bfloat16)
    w2t = W2.T.astype(jnp.bfloat16)
    scale = (jnp.exp(logit_scale) / math.sqrt(Ct)).reshape(1, 1)

    row = lambda a: a.reshape(1, -1)
    const = lambda shape: pl.BlockSpec(shape, lambda b: (0,) * len(shape))

    GROUPS = 1
    Bg = B // GROUPS
    ROWS_G = Bg * N

    info = plsc.get_sparse_core_info()
    n_tiles = info.num_cores * info.num_subcores
    rows_per_tile = ROWS_G // n_tiles
    chunk_rows = min(64, rows_per_tile)

    cp = pltpu.CompilerParams()
    if "needs_layout_passes" in pltpu.CompilerParams.__dataclass_fields__:
        cp = dataclasses.replace(cp, needs_layout_passes=False)
    mesh = plsc.VectorSubcoreMesh(core_axis_name="c", subcore_axis_name="s")
    sc_topk = pl.kernel(
        functools.partial(
            _sc_topk_kernel, rows_per_tile=rows_per_tile,
            chunk_rows=chunk_rows, k_pad=K_pad),
        mesh=mesh,
        out_type=[
            jax.ShapeDtypeStruct((ROWS_G, 16), jnp.float32),
            jax.ShapeDtypeStruct((ROWS_G, 16), jnp.int32),
        ],
        scratch_types=[
            pltpu.VMEM((2, chunk_rows, K_pad), jnp.float32),
            pltpu.VMEM((chunk_rows, 16), jnp.float32),
            pltpu.VMEM((chunk_rows, 16), jnp.int32),
            pltpu.SemaphoreType.DMA,
            pltpu.SemaphoreType.DMA,
        ],
        compiler_params=cp,
    )

    stage_a = pl.pallas_call(
        functools.partial(_stage_a_kernel, k_real=K),
        grid=(Bg,),
        in_specs=[
            pl.BlockSpec((1, N, Cv), lambda b: (b, 0, 0)),
            pl.BlockSpec((1, K_pad, Ct), lambda b: (b, 0, 0)),
            const((1, Cv)), const((1, Cv)),
            const((Cv, Ct)), const((1, 1)),
        ],
        out_specs=[
            pl.BlockSpec((1, N, Cv), lambda b: (b, 0, 0)),
            pl.BlockSpec((1, N, K_pad), lambda b: (b, 0, 0)),
        ],
        out_shape=[
            jax.ShapeDtypeStruct((Bg, N, Cv), jnp.bfloat16),
            jax.ShapeDtypeStruct((Bg, N, K_pad), jnp.float32),
        ],
    )

    stage_b = pl.pallas_call(
        functools.partial(_stage_b_kernel, k_pad=K_pad),
        grid=(Bg,),
        in_specs=[
            pl.BlockSpec((1, N, Cv), lambda b: (b, 0, 0)),
            pl.BlockSpec((1, K_pad, Ct), lambda b: (b, 0, 0)),
            pl.BlockSpec((1, N, 16), lambda b: (b, 0, 0)),
            pl.BlockSpec((1, N, 16), lambda b: (b, 0, 0)),
            const((Ct, Cv)), const((1, Cv)), const((1, 1)),
            const((1, Cv)), const((1, Cv)),
            const((Cv, dff)), const((1, dff)),
            const((dff, Cv)), const((1, Cv)),
        ],
        out_specs=pl.BlockSpec((1, N, Cv), lambda b: (b, 0, 0)),
        out_shape=jax.ShapeDtypeStruct((Bg, N, Cv), jnp.float32),
    )

    sel = []
    for g in range(GROUPS):
        sl = slice(g * Bg, (g + 1) * Bg)
        x_g, sim_g = stage_a(vis[sl], text_p[sl], row(g1), row(b1), wqt,
                             scale)
        w_g, i_g = sc_topk(sim_g.reshape(ROWS_G, K_pad))
        sel.append((x_g, w_g, i_g))

    outs = []
    for g in range(GROUPS):
        sl = slice(g * Bg, (g + 1) * Bg)
        x_g, w_g, i_g = sel[g]
        outs.append(stage_b(
            x_g, text_p[sl], w_g.reshape(Bg, N, 16), i_g.reshape(Bg, N, 16),
            wvt, row(Wg), bg.reshape(1, 1), row(g2), row(b2),
            w1t, row(bf1), w2t, row(bf2),
        ))
    out = jnp.concatenate(outs, axis=0)
    return out.reshape(B, H, W, Cv)


# speedup vs baseline: 1.0919x; 1.0568x over previous
import dataclasses
import functools
import math

import jax
import jax.numpy as jnp
from jax import lax
from jax.experimental import pallas as pl
from jax.experimental.pallas import tpu as pltpu
from jax.experimental.pallas import tpu_sc as plsc


def _dot(a, b):
    return jax.lax.dot_general(
        a, b, (((1,), (0,)), ((), ())), preferred_element_type=jnp.float32
    )


def _dot_t(a, b):
    return jax.lax.dot_general(
        a, b, (((1,), (1,)), ((), ())), preferred_element_type=jnp.float32
    )


def _ln_f32(x, g, b):
    m = jnp.mean(x, axis=1, keepdims=True)
    xc = x - m
    var = jnp.mean(xc * xc, axis=1, keepdims=True)
    return xc * jax.lax.rsqrt(var + 1e-5) * g + b


def _stage_a_kernel(vis_ref, text_ref, g1_ref, b1_ref, wq_ref, scale_ref,
                    x_ref, sim_ref, *, k_real):
    f32 = jnp.float32
    bf16 = jnp.bfloat16
    xv = vis_ref[0]
    x = _ln_f32(xv, g1_ref[0], b1_ref[0])
    x_ref[0] = x.astype(jnp.bfloat16)

    qpre = _dot(x.astype(bf16), wq_ref[...])
    qn = jnp.sqrt(jnp.sum(qpre * qpre, axis=1, keepdims=True))
    q = qpre / jnp.maximum(qn, 1e-12)

    t = text_ref[0]
    kn = jnp.sqrt(jnp.sum(t * t, axis=1, keepdims=True))
    k = t / jnp.maximum(kn, 1e-12)
    sim = _dot_t(q.astype(bf16), k.astype(bf16)) * scale_ref[0, 0]
    col = jax.lax.broadcasted_iota(jnp.int32, sim.shape, 1)
    sim_ref[0] = jnp.where(col < k_real, sim, jnp.asarray(-1e30, f32))


def _sc_merge(av, ai, bv, bi):
    rv = jnp.flip(bv, 0)
    ri = jnp.flip(bi, 0)
    ge = av >= rv
    mv = jnp.where(ge, av, rv)
    mi = jnp.where(ge, ai, ri)
    return plsc.sort_key_val(mv, mi, descending=True)


def _sc_topk_kernel(sim_hbm, wm_hbm, sim_v, wm_v, sem0, sem1, sem2, sem3, *,
                    rows_per_tile, chunk_rows, k_pad):
    f32 = jnp.float32
    i32 = jnp.int32
    n_chunks = k_pad // 16
    n_slabs = rows_per_tile // chunk_rows
    wid = lax.axis_index("s") * 2 + lax.axis_index("c")
    base = wid * rows_per_tile
    lane = lax.iota(i32, 16)
    in_sems = (sem0, sem1)
    out_sems = (sem2, sem3)
    zero16 = jnp.zeros((16,), f32)

    def slab_in(s, buf):
        return pltpu.make_async_copy(
            sim_hbm.at[pl.ds(base + s * chunk_rows, chunk_rows)],
            sim_v.at[buf], in_sems[buf])

    def slab_out(s, buf):
        return pltpu.make_async_copy(
            wm_v.at[buf],
            wm_hbm.at[pl.ds(base + s * chunk_rows, chunk_rows)],
            out_sems[buf])

    slab_in(0, 0).start()
    for s in range(n_slabs):
        buf = s % 2
        slab_in(s, buf).wait()
        if s + 1 < n_slabs:
            slab_in(s + 1, 1 - buf).start()
        if s >= 2:
            slab_out(s - 2, buf).wait()

        @plsc.parallel_loop(0, chunk_rows, unroll=2)
        def _(r):
            pairs = []
            for c in range(n_chunks):
                vals = sim_v[buf, r, pl.ds(c * 16, 16)]
                sv, si = plsc.sort_key_val(vals, lane + (c * 16),
                                           descending=True)
                pairs.append((sv, si))
            while len(pairs) > 1:
                nxt = []
                for j in range(0, len(pairs) - 1, 2):
                    nxt.append(_sc_merge(*pairs[j], *pairs[j + 1]))
                if len(pairs) % 2:
                    nxt.append(pairs[-1])
                pairs = nxt
            tv, ti = pairs[0]

            m0 = jnp.max(tv)
            e = jnp.exp(tv - jnp.full((16,), m0, f32))
            e = jnp.where(lane < 5, e, zero16)
            sm = jnp.sum(e)
            w = e / jnp.full((16,), sm, f32)
            for c in range(n_chunks):
                wm_v[buf, r, pl.ds(c * 16, 16)] = zero16
            plsc.store_scatter(wm_v, [jnp.full((16,), buf, i32),
                                      jnp.full((16,), r, i32), ti],
                               w, mask=lane < 5)

        slab_out(s, buf).start()

    for s in (n_slabs - 2, n_slabs - 1):
        if s >= 0:
            slab_out(s, s % 2).wait()


def _stage_b_kernel(x_ref, text_ref, wm_ref, wvt_ref, wg_ref, bg_ref,
                    g2_ref, b2_ref, w1t_ref, bf1_ref, w2t_ref, bf2_ref,
                    out_ref, *, k_pad):
    f32 = jnp.float32
    bf16 = jnp.bfloat16
    x = x_ref[0].astype(f32)
    wmat = wm_ref[0]

    t = text_ref[0]
    v = _dot(t.astype(bf16), wvt_ref[...]).astype(bf16)
    aligned = _dot(wmat.astype(bf16), v)

    gl = jnp.sum(x * wg_ref[0], axis=1, keepdims=True) + bg_ref[0, 0]
    gate = jax.nn.sigmoid(gl)
    y = x + aligned * gate
    y2 = _ln_f32(y, g2_ref[0], b2_ref[0])

    h = _dot(y2.astype(bf16), w1t_ref[...]) + bf1_ref[0]
    h = 0.5 * h * (1.0 + jax.lax.erf(h * jnp.asarray(0.7071067811865476, f32)))
    out_ref[0] = y2 + _dot(h.astype(bf16), w2t_ref[...]) + bf2_ref[0]


def kernel(visual_features, text_features, g1, b1, Wq, Wv, logit_scale,
           Wg, bg, g2, b2, W1, bf1, W2, bf2):
    B, H, W, Cv = visual_features.shape
    K, Ct = text_features.shape[1], text_features.shape[2]
    N = H * W
    dff = W1.shape[0]
    K_pad = ((K + 127) // 128) * 128
    ROWS = B * N

    text_p = jnp.concatenate(
        [text_features,
         jnp.zeros((B, K_pad - K, Ct), text_features.dtype)], axis=1)
    vis = visual_features.reshape(B, N, Cv)
    wqt = Wq.T.astype(jnp.bfloat16)
    wvt = Wv.T.astype(jnp.bfloat16)
    w1t = W1.T.astype(jnp.bfloat16)
    w2t = W2.T.astype(jnp.bfloat16)
    scale = (jnp.exp(logit_scale) / math.sqrt(Ct)).reshape(1, 1)

    row = lambda a: a.reshape(1, -1)
    const = lambda shape: pl.BlockSpec(shape, lambda b: (0,) * len(shape))

    GROUPS = 1
    Bg = B // GROUPS
    ROWS_G = Bg * N

    info = plsc.get_sparse_core_info()
    n_tiles = info.num_cores * info.num_subcores
    rows_per_tile = ROWS_G // n_tiles
    chunk_rows = min(64, rows_per_tile)

    cp = pltpu.CompilerParams()
    if "needs_layout_passes" in pltpu.CompilerParams.__dataclass_fields__:
        cp = dataclasses.replace(cp, needs_layout_passes=False)
    mesh = plsc.VectorSubcoreMesh(core_axis_name="c", subcore_axis_name="s")
    sc_topk = pl.kernel(
        functools.partial(
            _sc_topk_kernel, rows_per_tile=rows_per_tile,
            chunk_rows=chunk_rows, k_pad=K_pad),
        mesh=mesh,
        out_type=jax.ShapeDtypeStruct((ROWS_G, K_pad), jnp.float32),
        scratch_types=[
            pltpu.VMEM((2, chunk_rows, K_pad), jnp.float32),
            pltpu.VMEM((2, chunk_rows, K_pad), jnp.float32),
            pltpu.SemaphoreType.DMA,
            pltpu.SemaphoreType.DMA,
            pltpu.SemaphoreType.DMA,
            pltpu.SemaphoreType.DMA,
        ],
        compiler_params=cp,
    )

    stage_a = pl.pallas_call(
        functools.partial(_stage_a_kernel, k_real=K),
        grid=(Bg,),
        in_specs=[
            pl.BlockSpec((1, N, Cv), lambda b: (b, 0, 0)),
            pl.BlockSpec((1, K_pad, Ct), lambda b: (b, 0, 0)),
            const((1, Cv)), const((1, Cv)),
            const((Cv, Ct)), const((1, 1)),
        ],
        out_specs=[
            pl.BlockSpec((1, N, Cv), lambda b: (b, 0, 0)),
            pl.BlockSpec((1, N, K_pad), lambda b: (b, 0, 0)),
        ],
        out_shape=[
            jax.ShapeDtypeStruct((Bg, N, Cv), jnp.bfloat16),
            jax.ShapeDtypeStruct((Bg, N, K_pad), jnp.float32),
        ],
    )

    stage_b = pl.pallas_call(
        functools.partial(_stage_b_kernel, k_pad=K_pad),
        grid=(Bg,),
        in_specs=[
            pl.BlockSpec((1, N, Cv), lambda b: (b, 0, 0)),
            pl.BlockSpec((1, K_pad, Ct), lambda b: (b, 0, 0)),
            pl.BlockSpec((1, N, K_pad), lambda b: (b, 0, 0)),
            const((Ct, Cv)), const((1, Cv)), const((1, 1)),
            const((1, Cv)), const((1, Cv)),
            const((Cv, dff)), const((1, dff)),
            const((dff, Cv)), const((1, Cv)),
        ],
        out_specs=pl.BlockSpec((1, N, Cv), lambda b: (b, 0, 0)),
        out_shape=jax.ShapeDtypeStruct((Bg, N, Cv), jnp.float32),
    )

    sel = []
    for g in range(GROUPS):
        sl = slice(g * Bg, (g + 1) * Bg)
        x_g, sim_g = stage_a(vis[sl], text_p[sl], row(g1), row(b1), wqt,
                             scale)
        wm_g = sc_topk(sim_g.reshape(ROWS_G, K_pad))
        sel.append((x_g, wm_g))

    outs = []
    for g in range(GROUPS):
        sl = slice(g * Bg, (g + 1) * Bg)
        x_g, wm_g = sel[g]
        outs.append(stage_b(
            x_g, text_p[sl], wm_g.reshape(Bg, N, K_pad),
            wvt, row(Wg), bg.reshape(1, 1), row(g2), row(b2),
            w1t, row(bf1), w2t, row(bf2),
        ))
    out = jnp.concatenate(outs, axis=0)
    return out.reshape(B, H, W, Cv)
